# baseline (speedup 1.0000x reference)
import functools

import jax
import jax.numpy as jnp
from jax import lax
from jax.experimental import pallas as pl
from jax.experimental.pallas import tpu as pltpu
from jax.experimental.pallas import tpu_sc as plsc

_N = 10000
_E = 320000
_D_IN = 128
_H1 = 100
_F1 = 128
_H2 = 60
_FW = 64
_N_CLS = 4

_NC = 2
_NS = 16
_NW = _NC * _NS
_CH = 128
_NBUF = 5
_CHUNKS = 80
_EPW = _CHUNKS * _CH
_E_PAD = _NW * _EPW
_RPT = 632
_NH = _NS * _RPT

_mesh = plsc.VectorSubcoreMesh(core_axis_name="c", subcore_axis_name="s")

_STRIPE_CHUNKS = [(0, _CH), (_CH, _CH), (2 * _CH, _CH), (3 * _CH, _CH),
                  (4 * _CH, _RPT - 4 * _CH)]


def _deg_body(dsts_hbm, zeros_hbm, out_hbm, idxd_v, ones_v, zbuf, hist, dsem):
  c = lax.axis_index("c")
  s = lax.axis_index("s")
  w = s * _NC + c
  pltpu.sync_copy(zeros_hbm, zbuf)
  for off, sz in _STRIPE_CHUNKS:
    pltpu.sync_copy(zbuf.at[pl.ds(0, sz)],
                    hist.at[pl.ds(s * _RPT + off, sz)])
  pltpu.sync_copy(dsts_hbm.at[w], idxd_v)
  one = jnp.ones((16,), jnp.float32)
  for i in range(_CH // 16):
    ones_v[pl.ds(i * 16, 16)] = one
  plsc.subcore_barrier()

  def chunk(cc, carry):
    pltpu.async_copy(ones_v, hist.at[idxd_v.at[cc]], dsem, add=True)
    return carry

  lax.fori_loop(0, _CHUNKS, chunk, 0)

  def drain(cc, carry):
    pltpu.make_async_copy(ones_v, hist.at[idxd_v.at[0]], dsem).wait()
    return carry

  lax.fori_loop(0, _CHUNKS, drain, 0)
  plsc.subcore_barrier()
  for off, sz in _STRIPE_CHUNKS:
    pltpu.sync_copy(hist.at[pl.ds(s * _RPT + off, sz)], zbuf.at[pl.ds(0, sz)])
    pltpu.sync_copy(zbuf.at[pl.ds(0, sz)],
                    out_hbm.at[pl.ds(c * _NH + s * _RPT + off, sz)])


_deg_kernel = functools.partial(
    pl.kernel,
    out_type=jax.ShapeDtypeStruct((_NC * _NH,), jnp.float32),
    mesh=_mesh,
    scratch_types=[
        pltpu.VMEM((_CHUNKS, _CH), jnp.int32),
        pltpu.VMEM((_CH,), jnp.float32),
        pltpu.VMEM((_CH,), jnp.float32),
        pltpu.VMEM_SHARED((_NH,), jnp.float32),
        pltpu.SemaphoreType.DMA,
    ],
)(_deg_body)


def _seg_sum_body(y_hbm, srcs_hbm, dsts_hbm, zeros_hbm, out_hbm,
                  idxs_v, idxd_v, rows_v, acc, *sems):
  gsem = sems[:_NBUF]
  ssem = sems[_NBUF:]
  c = lax.axis_index("c")
  s = lax.axis_index("s")
  w = s * _NC + c
  pltpu.sync_copy(zeros_hbm, rows_v.at[0])
  for off, sz in _STRIPE_CHUNKS:
    pltpu.sync_copy(rows_v.at[0].at[pl.ds(0, sz)],
                    acc.at[pl.ds(s * _RPT + off, sz)])
  pltpu.sync_copy(srcs_hbm.at[w], idxs_v)
  pltpu.sync_copy(dsts_hbm.at[w], idxd_v)
  plsc.subcore_barrier()

  def sg(cc, b):
    pltpu.async_copy(y_hbm.at[idxs_v.at[cc]], rows_v.at[b], gsem[b])

  def wg(cc, b):
    pltpu.make_async_copy(y_hbm.at[idxs_v.at[cc]], rows_v.at[b],
                          gsem[b]).wait()

  def ss(cc, b):
    pltpu.async_copy(rows_v.at[b], acc.at[idxd_v.at[cc]], ssem[b], add=True)

  def ws(cc, b):
    pltpu.make_async_copy(rows_v.at[b], acc.at[idxd_v.at[cc]],
                          ssem[b]).wait()

  n_groups = _CHUNKS // _NBUF
  for b in range(_NBUF):
    sg(b, b)
  for b in range(_NBUF):
    wg(b, b)
    ss(b, b)
    if b > 0:
      ws(b - 1, b - 1)
      sg(b - 1 + _NBUF, b - 1)

  def group(g, carry):
    for b in range(_NBUF):
      cc = g * _NBUF + b
      pb = (b - 1) % _NBUF
      wg(cc, b)
      ss(cc, b)
      ws(cc - 1, pb)
      sg(cc - 1 + _NBUF, pb)
    return carry

  lax.fori_loop(1, n_groups - 1, group, 0)
  for b in range(_NBUF):
    cc = (n_groups - 1) * _NBUF + b
    pb = (b - 1) % _NBUF
    wg(cc, b)
    ss(cc, b)
    ws(cc - 1, pb)
    if cc - 1 + _NBUF < _CHUNKS:
      sg(cc - 1 + _NBUF, pb)
  ws(_CHUNKS - 1, (_CHUNKS - 1) % _NBUF)

  plsc.subcore_barrier()
  for off, sz in _STRIPE_CHUNKS:
    pltpu.sync_copy(acc.at[pl.ds(s * _RPT + off, sz)],
                    rows_v.at[0].at[pl.ds(0, sz)])
    pltpu.sync_copy(rows_v.at[0].at[pl.ds(0, sz)],
                    out_hbm.at[pl.ds(c * _NH + s * _RPT + off, sz)])


_seg_sum = functools.partial(
    pl.kernel,
    out_type=jax.ShapeDtypeStruct((_NC * _NH, _FW), jnp.float32),
    mesh=_mesh,
    compiler_params=pltpu.CompilerParams(use_tc_tiling_on_sc=False),
    scratch_types=(
        [pltpu.VMEM((_CHUNKS, _CH), jnp.int32),
         pltpu.VMEM((_CHUNKS, _CH), jnp.int32),
         pltpu.VMEM((_NBUF, _CH, _FW), jnp.float32),
         pltpu.VMEM_SHARED((_NH, _FW), jnp.float32)]
        + [pltpu.SemaphoreType.DMA] * (2 * _NBUF)
    ),
)(_seg_sum_body)


def _tc1_body(cnt_ref, x_ref, w1_ref, dis_ref, y1a_ref, y1b_ref):
  cnt = cnt_ref[...]
  deg = cnt[:_NH] + cnt[_NH:] + 1.0
  dis = lax.rsqrt(jnp.maximum(deg, 1.0))
  dis_ref[...] = dis
  xw = jnp.dot(x_ref[...], w1_ref[...], preferred_element_type=jnp.float32)
  y1 = dis[:_N, None] * xw
  y1a_ref[...] = y1[:, :_FW]
  y1b_ref[...] = y1[:, _FW:]


def _tc2_body(acca_ref, accb_ref, y1a_ref, y1b_ref, dis_ref, b1_ref, w2_ref,
              y2_ref):
  dis = dis_ref[...]
  aa = acca_ref[0:_N, :] + acca_ref[_NH:_NH + _N, :] + y1a_ref[...]
  ab = accb_ref[0:_N, :] + accb_ref[_NH:_NH + _N, :] + y1b_ref[...]
  a = jnp.concatenate([aa, ab], axis=1)
  h1 = jax.nn.relu(dis[:_N, None] * a + b1_ref[...][None, :])
  xw = jnp.dot(h1, w2_ref[...], preferred_element_type=jnp.float32)
  y2_ref[...] = dis[:_N, None] * xw


def _tc3_body(acc_ref, y2_ref, dis_ref, b2_ref, wc_ref, bc_ref, out_ref):
  dis = dis_ref[...]
  a = acc_ref[0:_N, :] + acc_ref[_NH:_NH + _N, :] + y2_ref[...]
  h2 = jax.nn.relu(dis[:_N, None] * a + b2_ref[...][None, :])
  logits = jnp.dot(h2, wc_ref[...], preferred_element_type=jnp.float32)
  logits = logits + bc_ref[...][None, :]
  m = jnp.max(logits, axis=1, keepdims=True)
  z = logits - m
  lse = jnp.log(jnp.sum(jnp.exp(z), axis=1, keepdims=True))
  out_ref[...] = z - lse


_tc1 = pl.pallas_call(
    _tc1_body,
    out_shape=(jax.ShapeDtypeStruct((_NH,), jnp.float32),
               jax.ShapeDtypeStruct((_N, _FW), jnp.float32),
               jax.ShapeDtypeStruct((_N, _FW), jnp.float32)),
)

_tc2 = pl.pallas_call(
    _tc2_body,
    out_shape=jax.ShapeDtypeStruct((_N, _FW), jnp.float32),
)

_tc3 = pl.pallas_call(
    _tc3_body,
    out_shape=jax.ShapeDtypeStruct((_N, _N_CLS), jnp.float32),
)


def kernel(x, edge_index, W1, b1, W2, b2, Wc, bc):
  src = edge_index[0].astype(jnp.int32)
  dst = edge_index[1].astype(jnp.int32)
  pad = _E_PAD - _E
  pad_src = (jnp.arange(pad, dtype=jnp.int32) * 37) % _N
  pad_dst = _N + jnp.arange(pad, dtype=jnp.int32) % (_NH - _N)
  srcs = jnp.concatenate([src, pad_src]).reshape(_NW, _CHUNKS, _CH)
  dsts = jnp.concatenate([dst, pad_dst]).reshape(_NW, _CHUNKS, _CH)

  zeros1 = jnp.zeros((_CH,), jnp.float32)
  zeros_f = jnp.zeros((_CH, _FW), jnp.float32)

  w1p = jnp.pad(W1, ((0, 0), (0, _F1 - _H1)))
  b1p = jnp.pad(b1, (0, _F1 - _H1))
  w2p = jnp.pad(W2, ((0, _F1 - _H1), (0, _FW - _H2)))
  b2p = jnp.pad(b2, (0, _FW - _H2))
  wcp = jnp.pad(Wc, ((0, _FW - _H2), (0, 0)))

  cnt = _deg_kernel(dsts, zeros1)
  dis, y1a, y1b = _tc1(cnt, x, w1p)
  acc1a = _seg_sum(y1a, srcs, dsts, zeros_f)
  acc1b = _seg_sum(y1b, srcs, dsts, zeros_f)
  y2 = _tc2(acc1a, acc1b, y1a, y1b, dis, b1p, w2p)
  acc2 = _seg_sum(y2, srcs, dsts, zeros_f)
  return _tc3(acc2, y2, dis, b2p, wcp, bc)


# speedup vs baseline: 1.0187x; 1.0187x over previous
import functools

import jax
import jax.numpy as jnp
from jax import lax
from jax.experimental import pallas as pl
from jax.experimental.pallas import tpu as pltpu
from jax.experimental.pallas import tpu_sc as plsc

_N = 10000
_E = 320000
_D_IN = 128
_H1 = 100
_F1 = 128
_H2 = 60
_FW = 64
_N_CLS = 4

_NC = 2
_NS = 16
_NW = _NC * _NS
_CH = 128
_NBUF = 5
_CHUNKS = 80
_EPW = _CHUNKS * _CH
_E_PAD = _NW * _EPW
_RPT = 632
_NH = _NS * _RPT

_mesh = plsc.VectorSubcoreMesh(core_axis_name="c", subcore_axis_name="s")

_STRIPE_CHUNKS = []
_off = 0
while _off < _RPT:
  _STRIPE_CHUNKS.append((_off, min(_CH, _RPT - _off)))
  _off += _CH


def _deg_body(dsts_hbm, zeros_hbm, out_hbm, idxd_v, ones_v, zbuf, hist, dsem):
  c = lax.axis_index("c")
  s = lax.axis_index("s")
  w = s * _NC + c
  pltpu.sync_copy(zeros_hbm, zbuf)
  for off, sz in _STRIPE_CHUNKS:
    pltpu.sync_copy(zbuf.at[pl.ds(0, sz)],
                    hist.at[pl.ds(s * _RPT + off, sz)])
  pltpu.sync_copy(dsts_hbm.at[w], idxd_v)
  one = jnp.ones((16,), jnp.float32)
  for i in range(_CH // 16):
    ones_v[pl.ds(i * 16, 16)] = one
  plsc.subcore_barrier()

  def chunk(cc, carry):
    pltpu.async_copy(ones_v, hist.at[idxd_v.at[cc]], dsem, add=True)
    return carry

  lax.fori_loop(0, _CHUNKS, chunk, 0)

  def drain(cc, carry):
    pltpu.make_async_copy(ones_v, hist.at[idxd_v.at[0]], dsem).wait()
    return carry

  lax.fori_loop(0, _CHUNKS, drain, 0)
  plsc.subcore_barrier()
  for off, sz in _STRIPE_CHUNKS:
    pltpu.sync_copy(hist.at[pl.ds(s * _RPT + off, sz)], zbuf.at[pl.ds(0, sz)])
    pltpu.sync_copy(zbuf.at[pl.ds(0, sz)],
                    out_hbm.at[pl.ds(c * _NH + s * _RPT + off, sz)])


_deg_kernel = functools.partial(
    pl.kernel,
    out_type=jax.ShapeDtypeStruct((_NC * _NH,), jnp.float32),
    mesh=_mesh,
    scratch_types=[
        pltpu.VMEM((_CHUNKS, _CH), jnp.int32),
        pltpu.VMEM((_CH,), jnp.float32),
        pltpu.VMEM((_CH,), jnp.float32),
        pltpu.VMEM_SHARED((_NH,), jnp.float32),
        pltpu.SemaphoreType.DMA,
    ],
)(_deg_body)


def _seg_sum_body(y_hbm, srcs_hbm, dsts_hbm, zeros_hbm, out_hbm,
                  idxs_v, idxd_v, rows_v, acc, *sems):
  gsem = sems[:_NBUF]
  ssem = sems[_NBUF:]
  c = lax.axis_index("c")
  s = lax.axis_index("s")
  w = s * _NC + c
  pltpu.sync_copy(zeros_hbm, rows_v.at[0])
  for off, sz in _STRIPE_CHUNKS:
    pltpu.sync_copy(rows_v.at[0].at[pl.ds(0, sz)],
                    acc.at[pl.ds(s * _RPT + off, sz)])
  pltpu.sync_copy(srcs_hbm.at[w], idxs_v)
  pltpu.sync_copy(dsts_hbm.at[w], idxd_v)
  plsc.subcore_barrier()

  def sg(cc, b):
    pltpu.async_copy(y_hbm.at[idxs_v.at[cc]], rows_v.at[b], gsem[b])

  def wg(cc, b):
    pltpu.make_async_copy(y_hbm.at[idxs_v.at[cc]], rows_v.at[b],
                          gsem[b]).wait()

  def ss(cc, b):
    pltpu.async_copy(rows_v.at[b], acc.at[idxd_v.at[cc]], ssem[b], add=True)

  def ws(cc, b):
    pltpu.make_async_copy(rows_v.at[b], acc.at[idxd_v.at[cc]],
                          ssem[b]).wait()

  for b in range(_NBUF):
    sg(b, b)

  def group(g, carry):
    for b in range(_NBUF):
      cc = g * _NBUF + b
      wg(cc, b)
      ss(cc, b)
      ws(cc, b)
      sg(cc + _NBUF, b)
    return carry

  n_groups = _CHUNKS // _NBUF
  lax.fori_loop(0, n_groups - 1, group, 0)
  for b in range(_NBUF):
    cc = (n_groups - 1) * _NBUF + b
    wg(cc, b)
    ss(cc, b)
    ws(cc, b)

  plsc.subcore_barrier()
  for off, sz in _STRIPE_CHUNKS:
    pltpu.sync_copy(acc.at[pl.ds(s * _RPT + off, sz)],
                    rows_v.at[0].at[pl.ds(0, sz)])
    pltpu.sync_copy(rows_v.at[0].at[pl.ds(0, sz)],
                    out_hbm.at[pl.ds(c * _NH + s * _RPT + off, sz)])


_seg_sum = functools.partial(
    pl.kernel,
    out_type=jax.ShapeDtypeStruct((_NC * _NH, _FW), jnp.float32),
    mesh=_mesh,
    compiler_params=pltpu.CompilerParams(use_tc_tiling_on_sc=False),
    scratch_types=(
        [pltpu.VMEM((_CHUNKS, _CH), jnp.int32),
         pltpu.VMEM((_CHUNKS, _CH), jnp.int32),
         pltpu.VMEM((_NBUF, _CH, _FW), jnp.float32),
         pltpu.VMEM_SHARED((_NH, _FW), jnp.float32)]
        + [pltpu.SemaphoreType.DMA] * (2 * _NBUF)
    ),
)(_seg_sum_body)


def _tc1_body(cnt_ref, x_ref, w1_ref, dis_ref, y1a_ref, y1b_ref):
  cnt = cnt_ref[...]
  deg = cnt[:_NH] + cnt[_NH:] + 1.0
  dis = lax.rsqrt(jnp.maximum(deg, 1.0))
  dis_ref[...] = dis
  xw = jnp.dot(x_ref[...], w1_ref[...], preferred_element_type=jnp.float32)
  y1 = dis[:_N, None] * xw
  y1a_ref[...] = y1[:, :_FW]
  y1b_ref[...] = y1[:, _FW:]


def _tc2_body(acca_ref, accb_ref, y1a_ref, y1b_ref, dis_ref, b1_ref, w2_ref,
              y2_ref):
  dis = dis_ref[...]
  aa = acca_ref[0:_N, :] + acca_ref[_NH:_NH + _N, :] + y1a_ref[...]
  ab = accb_ref[0:_N, :] + accb_ref[_NH:_NH + _N, :] + y1b_ref[...]
  a = jnp.concatenate([aa, ab], axis=1)
  h1 = jax.nn.relu(dis[:_N, None] * a + b1_ref[...][None, :])
  xw = jnp.dot(h1, w2_ref[...], preferred_element_type=jnp.float32)
  y2_ref[...] = dis[:_N, None] * xw


def _tc3_body(acc_ref, y2_ref, dis_ref, b2_ref, wc_ref, bc_ref, out_ref):
  dis = dis_ref[...]
  a = acc_ref[0:_N, :] + acc_ref[_NH:_NH + _N, :] + y2_ref[...]
  h2 = jax.nn.relu(dis[:_N, None] * a + b2_ref[...][None, :])
  logits = jnp.dot(h2, wc_ref[...], preferred_element_type=jnp.float32)
  logits = logits + bc_ref[...][None, :]
  m = jnp.max(logits, axis=1, keepdims=True)
  z = logits - m
  lse = jnp.log(jnp.sum(jnp.exp(z), axis=1, keepdims=True))
  out_ref[...] = z - lse


_tc1 = pl.pallas_call(
    _tc1_body,
    out_shape=(jax.ShapeDtypeStruct((_NH,), jnp.float32),
               jax.ShapeDtypeStruct((_N, _FW), jnp.float32),
               jax.ShapeDtypeStruct((_N, _FW), jnp.float32)),
)

_tc2 = pl.pallas_call(
    _tc2_body,
    out_shape=jax.ShapeDtypeStruct((_N, _FW), jnp.float32),
)

_tc3 = pl.pallas_call(
    _tc3_body,
    out_shape=jax.ShapeDtypeStruct((_N, _N_CLS), jnp.float32),
)


def kernel(x, edge_index, W1, b1, W2, b2, Wc, bc):
  src = edge_index[0].astype(jnp.int32)
  dst = edge_index[1].astype(jnp.int32)
  pad = _E_PAD - _E
  pad_src = (jnp.arange(pad, dtype=jnp.int32) * 37) % _N
  pad_dst = _N + jnp.arange(pad, dtype=jnp.int32) % (_NH - _N)
  srcs = jnp.concatenate([src, pad_src]).reshape(_NW, _CHUNKS, _CH)
  dsts = jnp.concatenate([dst, pad_dst]).reshape(_NW, _CHUNKS, _CH)

  zeros1 = jnp.zeros((_CH,), jnp.float32)
  zeros_f = jnp.zeros((_CH, _FW), jnp.float32)

  w1p = jnp.pad(W1, ((0, 0), (0, _F1 - _H1)))
  b1p = jnp.pad(b1, (0, _F1 - _H1))
  w2p = jnp.pad(W2, ((0, _F1 - _H1), (0, _FW - _H2)))
  b2p = jnp.pad(b2, (0, _FW - _H2))
  wcp = jnp.pad(Wc, ((0, _FW - _H2), (0, 0)))

  cnt = _deg_kernel(dsts, zeros1)
  dis, y1a, y1b = _tc1(cnt, x, w1p)
  acc1a = _seg_sum(y1a, srcs, dsts, zeros_f)
  acc1b = _seg_sum(y1b, srcs, dsts, zeros_f)
  y2 = _tc2(acc1a, acc1b, y1a, y1b, dis, b1p, w2p)
  acc2 = _seg_sum(y2, srcs, dsts, zeros_f)
  return _tc3(acc2, y2, dis, b2p, wcp, bc)


# speedup vs baseline: 1.3748x; 1.3495x over previous
import functools

import jax
import jax.numpy as jnp
from jax import lax
from jax.experimental import pallas as pl
from jax.experimental.pallas import tpu as pltpu
from jax.experimental.pallas import tpu_sc as plsc

_N = 10000
_E = 320000
_D_IN = 128
_H1 = 100
_F1 = 128
_H2 = 60
_FW = 64
_N_CLS = 4

_NC = 2
_NS = 16
_NW = _NC * _NS
_CH = 128
_NBUF = 5
_CHUNKS = 80
_EPW = _CHUNKS * _CH
_E_PAD = _NW * _EPW
_RPT = 632
_NH = _NS * _RPT

_mesh = plsc.VectorSubcoreMesh(core_axis_name="c", subcore_axis_name="s")

_STRIPE_CHUNKS = []
_off = 0
while _off < _RPT:
  _STRIPE_CHUNKS.append((_off, min(_CH, _RPT - _off)))
  _off += _CH


def _deg_body(dsts_hbm, zeros_hbm, out_hbm, idxd_v, ones_v, zbuf, hist, dsem):
  c = lax.axis_index("c")
  s = lax.axis_index("s")
  w = s * _NC + c
  pltpu.sync_copy(zeros_hbm, zbuf)
  for off, sz in _STRIPE_CHUNKS:
    pltpu.sync_copy(zbuf.at[pl.ds(0, sz)],
                    hist.at[pl.ds(s * _RPT + off, sz)])
  pltpu.sync_copy(dsts_hbm.at[w], idxd_v)
  one = jnp.ones((16,), jnp.float32)
  for i in range(_CH // 16):
    ones_v[pl.ds(i * 16, 16)] = one
  plsc.subcore_barrier()

  def chunk(cc, carry):
    pltpu.async_copy(ones_v, hist.at[idxd_v.at[cc]], dsem, add=True)
    return carry

  lax.fori_loop(0, _CHUNKS, chunk, 0)

  def drain(cc, carry):
    pltpu.make_async_copy(ones_v, hist.at[idxd_v.at[0]], dsem).wait()
    return carry

  lax.fori_loop(0, _CHUNKS, drain, 0)
  plsc.subcore_barrier()
  for off, sz in _STRIPE_CHUNKS:
    pltpu.sync_copy(hist.at[pl.ds(s * _RPT + off, sz)], zbuf.at[pl.ds(0, sz)])
    pltpu.sync_copy(zbuf.at[pl.ds(0, sz)],
                    out_hbm.at[pl.ds(c * _NH + s * _RPT + off, sz)])


_deg_kernel = functools.partial(
    pl.kernel,
    out_type=jax.ShapeDtypeStruct((_NC * _NH,), jnp.float32),
    mesh=_mesh,
    scratch_types=[
        pltpu.VMEM((_CHUNKS, _CH), jnp.int32),
        pltpu.VMEM((_CH,), jnp.float32),
        pltpu.VMEM((_CH,), jnp.float32),
        pltpu.VMEM_SHARED((_NH,), jnp.float32),
        pltpu.SemaphoreType.DMA,
    ],
)(_deg_body)


def _seg_sum_body(y_hbm, srcs_hbm, dsts_hbm, zeros_hbm, out_hbm,
                  idxs_v, idxd_v, rows_v, acc, *sems):
  gsem = sems[:_NBUF]
  ssem = sems[_NBUF:]
  c = lax.axis_index("c")
  s = lax.axis_index("s")
  w = s * _NC + c
  pltpu.sync_copy(zeros_hbm, rows_v.at[0])
  for off, sz in _STRIPE_CHUNKS:
    pltpu.sync_copy(rows_v.at[0].at[pl.ds(0, sz)],
                    acc.at[pl.ds(s * _RPT + off, sz)])
  pltpu.sync_copy(srcs_hbm.at[w], idxs_v)
  pltpu.sync_copy(dsts_hbm.at[w], idxd_v)
  plsc.subcore_barrier()

  def sg(cc, b):
    pltpu.async_copy(y_hbm.at[idxs_v.at[cc]], rows_v.at[b], gsem[b])

  def wg(cc, b):
    pltpu.make_async_copy(y_hbm.at[idxs_v.at[cc]], rows_v.at[b],
                          gsem[b]).wait()

  def ss(cc, b):
    pltpu.async_copy(rows_v.at[b], acc.at[idxd_v.at[cc]], ssem[b], add=True)

  def ws(cc, b):
    pltpu.make_async_copy(rows_v.at[b], acc.at[idxd_v.at[cc]],
                          ssem[b]).wait()

  for b in range(_NBUF):
    sg(b, b)

  def group(g, carry):
    for b in range(_NBUF):
      cc = g * _NBUF + b
      wg(cc, b)
      ss(cc, b)
      ws(cc, b)
      sg(cc + _NBUF, b)
    return carry

  n_groups = _CHUNKS // _NBUF
  lax.fori_loop(0, n_groups - 1, group, 0)
  for b in range(_NBUF):
    cc = (n_groups - 1) * _NBUF + b
    wg(cc, b)
    ss(cc, b)
    ws(cc, b)

  plsc.subcore_barrier()
  for off, sz in _STRIPE_CHUNKS:
    pltpu.sync_copy(acc.at[pl.ds(s * _RPT + off, sz)],
                    rows_v.at[0].at[pl.ds(0, sz)])
    pltpu.sync_copy(rows_v.at[0].at[pl.ds(0, sz)],
                    out_hbm.at[pl.ds(c * _NH + s * _RPT + off, sz)])


def _make_seg(F):
  return functools.partial(
      pl.kernel,
      out_type=jax.ShapeDtypeStruct((_NC * _NH, F), jnp.bfloat16),
      mesh=_mesh,
      compiler_params=pltpu.CompilerParams(use_tc_tiling_on_sc=False),
      scratch_types=(
          [pltpu.VMEM((_CHUNKS, _CH), jnp.int32),
           pltpu.VMEM((_CHUNKS, _CH), jnp.int32),
           pltpu.VMEM((_NBUF, _CH, F), jnp.bfloat16),
           pltpu.VMEM_SHARED((_NH, F), jnp.bfloat16)]
          + [pltpu.SemaphoreType.DMA] * (2 * _NBUF)
      ),
  )(_seg_sum_body)


_seg_sum_f1 = _make_seg(_F1)
_seg_sum_f2 = _make_seg(_FW)


def _tc1_body(cnt_ref, x_ref, w1_ref, dis_ref, y1_ref):
  cnt = cnt_ref[...]
  deg = cnt[:_NH] + cnt[_NH:] + 1.0
  dis = lax.rsqrt(jnp.maximum(deg, 1.0))
  dis_ref[...] = dis
  xw = jnp.dot(x_ref[...], w1_ref[...], preferred_element_type=jnp.float32)
  y1_ref[...] = (dis[:_N, None] * xw).astype(jnp.bfloat16)


def _tc2_body(acc_ref, y1_ref, dis_ref, b1_ref, w2_ref, y2_ref):
  dis = dis_ref[...]
  a = (acc_ref[0:_N, :].astype(jnp.float32)
       + acc_ref[_NH:_NH + _N, :].astype(jnp.float32)
       + y1_ref[...].astype(jnp.float32))
  h1 = jax.nn.relu(dis[:_N, None] * a + b1_ref[...][None, :])
  xw = jnp.dot(h1, w2_ref[...], preferred_element_type=jnp.float32)
  y2_ref[...] = (dis[:_N, None] * xw).astype(jnp.bfloat16)


def _tc3_body(acc_ref, y2_ref, dis_ref, b2_ref, wc_ref, bc_ref, out_ref):
  dis = dis_ref[...]
  a = (acc_ref[0:_N, :].astype(jnp.float32)
       + acc_ref[_NH:_NH + _N, :].astype(jnp.float32)
       + y2_ref[...].astype(jnp.float32))
  h2 = jax.nn.relu(dis[:_N, None] * a + b2_ref[...][None, :])
  logits = jnp.dot(h2, wc_ref[...], preferred_element_type=jnp.float32)
  logits = logits + bc_ref[...][None, :]
  m = jnp.max(logits, axis=1, keepdims=True)
  z = logits - m
  lse = jnp.log(jnp.sum(jnp.exp(z), axis=1, keepdims=True))
  out_ref[...] = z - lse


_tc1 = pl.pallas_call(
    _tc1_body,
    out_shape=(jax.ShapeDtypeStruct((_NH,), jnp.float32),
               jax.ShapeDtypeStruct((_N, _F1), jnp.bfloat16)),
)

_tc2 = pl.pallas_call(
    _tc2_body,
    out_shape=jax.ShapeDtypeStruct((_N, _FW), jnp.bfloat16),
)

_tc3 = pl.pallas_call(
    _tc3_body,
    out_shape=jax.ShapeDtypeStruct((_N, _N_CLS), jnp.float32),
)


def kernel(x, edge_index, W1, b1, W2, b2, Wc, bc):
  src = edge_index[0].astype(jnp.int32)
  dst = edge_index[1].astype(jnp.int32)
  pad = _E_PAD - _E
  pad_src = (jnp.arange(pad, dtype=jnp.int32) * 37) % _N
  pad_dst = _N + jnp.arange(pad, dtype=jnp.int32) % (_NH - _N)
  srcs = jnp.concatenate([src, pad_src]).reshape(_NW, _CHUNKS, _CH)
  dsts = jnp.concatenate([dst, pad_dst]).reshape(_NW, _CHUNKS, _CH)

  zeros1 = jnp.zeros((_CH,), jnp.float32)
  zeros_f1 = jnp.zeros((_CH, _F1), jnp.bfloat16)
  zeros_f2 = jnp.zeros((_CH, _FW), jnp.bfloat16)

  w1p = jnp.pad(W1, ((0, 0), (0, _F1 - _H1)))
  b1p = jnp.pad(b1, (0, _F1 - _H1))
  w2p = jnp.pad(W2, ((0, _F1 - _H1), (0, _FW - _H2)))
  b2p = jnp.pad(b2, (0, _FW - _H2))
  wcp = jnp.pad(Wc, ((0, _FW - _H2), (0, 0)))

  cnt = _deg_kernel(dsts, zeros1)
  dis, y1 = _tc1(cnt, x, w1p)
  acc1 = _seg_sum_f1(y1, srcs, dsts, zeros_f1)
  y2 = _tc2(acc1, y1, dis, b1p, w2p)
  acc2 = _seg_sum_f2(y2, srcs, dsts, zeros_f2)
  return _tc3(acc2, y2, dis, b2p, wcp, bc)


# speedup vs baseline: 1.4106x; 1.0260x over previous
import functools

import jax
import jax.numpy as jnp
from jax import lax
from jax.experimental import pallas as pl
from jax.experimental.pallas import tpu as pltpu
from jax.experimental.pallas import tpu_sc as plsc

_N = 10000
_E = 320000
_D_IN = 128
_H1 = 100
_F1 = 112
_H2 = 60
_FW = 64
_N_CLS = 4

_NC = 2
_NS = 16
_NW = _NC * _NS
_CH = 128
_NBUF = 5
_CHUNKS = 80
_EPW = _CHUNKS * _CH
_E_PAD = _NW * _EPW
_RPT = 632
_NH = _NS * _RPT

_mesh = plsc.VectorSubcoreMesh(core_axis_name="c", subcore_axis_name="s")

_STRIPE_CHUNKS = []
_off = 0
while _off < _RPT:
  _STRIPE_CHUNKS.append((_off, min(_CH, _RPT - _off)))
  _off += _CH


def _deg_body(dsts_hbm, zeros_hbm, out_hbm, idxd_v, ones_v, zbuf, hist, dsem):
  c = lax.axis_index("c")
  s = lax.axis_index("s")
  w = s * _NC + c
  pltpu.sync_copy(zeros_hbm, zbuf)
  for off, sz in _STRIPE_CHUNKS:
    pltpu.sync_copy(zbuf.at[pl.ds(0, sz)],
                    hist.at[pl.ds(s * _RPT + off, sz)])
  pltpu.sync_copy(dsts_hbm.at[w], idxd_v)
  one = jnp.ones((16,), jnp.float32)
  for i in range(_CH // 16):
    ones_v[pl.ds(i * 16, 16)] = one
  plsc.subcore_barrier()

  def chunk(cc, carry):
    pltpu.async_copy(ones_v, hist.at[idxd_v.at[cc]], dsem, add=True)
    return carry

  lax.fori_loop(0, _CHUNKS, chunk, 0)

  def drain(cc, carry):
    pltpu.make_async_copy(ones_v, hist.at[idxd_v.at[0]], dsem).wait()
    return carry

  lax.fori_loop(0, _CHUNKS, drain, 0)
  plsc.subcore_barrier()
  for off, sz in _STRIPE_CHUNKS:
    pltpu.sync_copy(hist.at[pl.ds(s * _RPT + off, sz)], zbuf.at[pl.ds(0, sz)])
    pltpu.sync_copy(zbuf.at[pl.ds(0, sz)],
                    out_hbm.at[pl.ds(c * _NH + s * _RPT + off, sz)])


_deg_kernel = functools.partial(
    pl.kernel,
    out_type=jax.ShapeDtypeStruct((_NC * _NH,), jnp.float32),
    mesh=_mesh,
    scratch_types=[
        pltpu.VMEM((_CHUNKS, _CH), jnp.int32),
        pltpu.VMEM((_CH,), jnp.float32),
        pltpu.VMEM((_CH,), jnp.float32),
        pltpu.VMEM_SHARED((_NH,), jnp.float32),
        pltpu.SemaphoreType.DMA,
    ],
)(_deg_body)


def _seg_sum_body(y_hbm, srcs_hbm, dsts_hbm, zeros_hbm, out_hbm,
                  idxs_v, idxd_v, rows_v, acc, *sems):
  gsem = sems[:_NBUF]
  ssem = sems[_NBUF:]
  c = lax.axis_index("c")
  s = lax.axis_index("s")
  w = s * _NC + c
  pltpu.sync_copy(zeros_hbm, rows_v.at[0])
  for off, sz in _STRIPE_CHUNKS:
    pltpu.sync_copy(rows_v.at[0].at[pl.ds(0, sz)],
                    acc.at[pl.ds(s * _RPT + off, sz)])
  pltpu.sync_copy(srcs_hbm.at[w], idxs_v)
  pltpu.sync_copy(dsts_hbm.at[w], idxd_v)
  plsc.subcore_barrier()

  def sg(cc, b):
    pltpu.async_copy(y_hbm.at[idxs_v.at[cc]], rows_v.at[b], gsem[b])

  def wg(cc, b):
    pltpu.make_async_copy(y_hbm.at[idxs_v.at[cc]], rows_v.at[b],
                          gsem[b]).wait()

  def ss(cc, b):
    pltpu.async_copy(rows_v.at[b], acc.at[idxd_v.at[cc]], ssem[b], add=True)

  def ws(cc, b):
    pltpu.make_async_copy(rows_v.at[b], acc.at[idxd_v.at[cc]],
                          ssem[b]).wait()

  for b in range(_NBUF):
    sg(b, b)

  def group(g, carry):
    for b in range(_NBUF):
      cc = g * _NBUF + b
      wg(cc, b)
      ss(cc, b)
      ws(cc, b)
      sg(cc + _NBUF, b)
    return carry

  n_groups = _CHUNKS // _NBUF
  lax.fori_loop(0, n_groups - 1, group, 0)
  for b in range(_NBUF):
    cc = (n_groups - 1) * _NBUF + b
    wg(cc, b)
    ss(cc, b)
    ws(cc, b)

  plsc.subcore_barrier()
  for off, sz in _STRIPE_CHUNKS:
    pltpu.sync_copy(acc.at[pl.ds(s * _RPT + off, sz)],
                    rows_v.at[0].at[pl.ds(0, sz)])
    pltpu.sync_copy(rows_v.at[0].at[pl.ds(0, sz)],
                    out_hbm.at[pl.ds(c * _NH + s * _RPT + off, sz)])


def _make_seg(F):
  return functools.partial(
      pl.kernel,
      out_type=jax.ShapeDtypeStruct((_NC * _NH, F), jnp.bfloat16),
      mesh=_mesh,
      compiler_params=pltpu.CompilerParams(use_tc_tiling_on_sc=False),
      scratch_types=(
          [pltpu.VMEM((_CHUNKS, _CH), jnp.int32),
           pltpu.VMEM((_CHUNKS, _CH), jnp.int32),
           pltpu.VMEM((_NBUF, _CH, F), jnp.bfloat16),
           pltpu.VMEM_SHARED((_NH, F), jnp.bfloat16)]
          + [pltpu.SemaphoreType.DMA] * (2 * _NBUF)
      ),
  )(_seg_sum_body)


_seg_sum_f1 = _make_seg(_F1)
_seg_sum_f2 = _make_seg(_FW)


def _tc1_body(cnt_ref, x_ref, w1_ref, dis_ref, y1_ref):
  cnt = cnt_ref[...]
  deg = cnt[:_NH] + cnt[_NH:] + 1.0
  dis = lax.rsqrt(jnp.maximum(deg, 1.0))
  dis_ref[...] = dis
  xw = jnp.dot(x_ref[...], w1_ref[...], preferred_element_type=jnp.float32)
  y1_ref[...] = (dis[:_N, None] * xw).astype(jnp.bfloat16)


def _tc2_body(acc_ref, y1_ref, dis_ref, b1_ref, w2_ref, y2_ref):
  dis = dis_ref[...]
  a = (acc_ref[0:_N, :].astype(jnp.float32)
       + acc_ref[_NH:_NH + _N, :].astype(jnp.float32)
       + y1_ref[...].astype(jnp.float32))
  h1 = jax.nn.relu(dis[:_N, None] * a + b1_ref[...][None, :])
  xw = jnp.dot(h1, w2_ref[...], preferred_element_type=jnp.float32)
  y2_ref[...] = (dis[:_N, None] * xw).astype(jnp.bfloat16)


def _tc3_body(acc_ref, y2_ref, dis_ref, b2_ref, wc_ref, bc_ref, out_ref):
  dis = dis_ref[...]
  a = (acc_ref[0:_N, :].astype(jnp.float32)
       + acc_ref[_NH:_NH + _N, :].astype(jnp.float32)
       + y2_ref[...].astype(jnp.float32))
  h2 = jax.nn.relu(dis[:_N, None] * a + b2_ref[...][None, :])
  logits = jnp.dot(h2, wc_ref[...], preferred_element_type=jnp.float32)
  logits = logits + bc_ref[...][None, :]
  m = jnp.max(logits, axis=1, keepdims=True)
  z = logits - m
  lse = jnp.log(jnp.sum(jnp.exp(z), axis=1, keepdims=True))
  out_ref[...] = z - lse


_tc1 = pl.pallas_call(
    _tc1_body,
    out_shape=(jax.ShapeDtypeStruct((_NH,), jnp.float32),
               jax.ShapeDtypeStruct((_N, _F1), jnp.bfloat16)),
)

_tc2 = pl.pallas_call(
    _tc2_body,
    out_shape=jax.ShapeDtypeStruct((_N, _FW), jnp.bfloat16),
)

_tc3 = pl.pallas_call(
    _tc3_body,
    out_shape=jax.ShapeDtypeStruct((_N, _N_CLS), jnp.float32),
)


def kernel(x, edge_index, W1, b1, W2, b2, Wc, bc):
  src = edge_index[0].astype(jnp.int32)
  dst = edge_index[1].astype(jnp.int32)
  pad = _E_PAD - _E
  pad_src = (jnp.arange(pad, dtype=jnp.int32) * 37) % _N
  pad_dst = _N + jnp.arange(pad, dtype=jnp.int32) % (_NH - _N)
  srcs = jnp.concatenate([src, pad_src]).reshape(_NW, _CHUNKS, _CH)
  dsts = jnp.concatenate([dst, pad_dst]).reshape(_NW, _CHUNKS, _CH)

  zeros1 = jnp.zeros((_CH,), jnp.float32)
  zeros_f1 = jnp.zeros((_CH, _F1), jnp.bfloat16)
  zeros_f2 = jnp.zeros((_CH, _FW), jnp.bfloat16)

  w1p = jnp.pad(W1, ((0, 0), (0, _F1 - _H1)))
  b1p = jnp.pad(b1, (0, _F1 - _H1))
  w2p = jnp.pad(W2, ((0, _F1 - _H1), (0, _FW - _H2)))
  b2p = jnp.pad(b2, (0, _FW - _H2))
  wcp = jnp.pad(Wc, ((0, _FW - _H2), (0, 0)))

  cnt = _deg_kernel(dsts, zeros1)
  dis, y1 = _tc1(cnt, x, w1p)
  acc1 = _seg_sum_f1(y1, srcs, dsts, zeros_f1)
  y2 = _tc2(acc1, y1, dis, b1p, w2p)
  acc2 = _seg_sum_f2(y2, srcs, dsts, zeros_f2)
  return _tc3(acc2, y2, dis, b2p, wcp, bc)
